# baseline (speedup 1.0000x reference)
import jax
import jax.numpy as jnp
from jax import lax
from jax.experimental import pallas as pl
from jax.experimental.pallas import tpu as pltpu, tpu_sc as plsc

D = 768
NC, NS, L = 2, 16, 16
NW = NC * NS
CH = 64


def _emb_body(n_tokens, seq_len, idx_hbm, table_hbm, pe_hbm, out_hbm,
              idx_v, rows_v, pe_v, sem):
    per_w = n_tokens // NW
    nchunk = per_w // CH
    wid = lax.axis_index("s") * NC + lax.axis_index("c")
    base = wid * per_w
    s0 = lax.rem(base, seq_len)
    for c in range(nchunk):
        pltpu.sync_copy(idx_hbm.at[pl.ds(base + c * CH, CH)], idx_v)
        gat = pltpu.async_copy(table_hbm.at[idx_v], rows_v, sem)
        pltpu.sync_copy(pe_hbm.at[pl.ds(s0 + c * CH, CH)], pe_v)
        gat.wait()

        def add_row(r, carry):
            for j in range(D // L):
                sl = pl.ds(j * L, L)
                rows_v[r, sl] = rows_v[r, sl] + pe_v[r, sl]
            return carry

        lax.fori_loop(0, CH, add_row, 0)
        pltpu.sync_copy(rows_v, out_hbm.at[pl.ds(base + c * CH, CH)])


def kernel(x, token_table, pe):
    B, S = x.shape
    n = B * S
    xf = x.reshape(-1).astype(jnp.int32)
    pe_s = pe[:S]
    mesh = plsc.VectorSubcoreMesh(core_axis_name="c", subcore_axis_name="s",
                                  num_cores=NC, num_subcores=NS)

    def body(*refs):
        _emb_body(n, S, *refs)

    out = pl.kernel(
        body,
        out_type=jax.ShapeDtypeStruct((n, D), jnp.float32),
        mesh=mesh,
        scratch_types=[
            pltpu.VMEM((CH,), jnp.int32),
            pltpu.VMEM((CH, D), jnp.float32),
            pltpu.VMEM((CH, D), jnp.float32),
            pltpu.SemaphoreType.DMA,
        ],
    )(xf, token_table, pe_s)
    return out.reshape(B, S, D)
